# Initial kernel scaffold; baseline (speedup 1.0000x reference)
import jax
import jax.numpy as jnp
from jax.experimental import pallas as pl


def kernel(token_ids, lookup):
    raise NotImplementedError("write your pallas kernel here")


# speedup vs baseline: 3.3308x; 3.3308x over previous
import functools

import jax
import jax.numpy as jnp
from jax import lax
from jax.experimental import pallas as pl
from jax.experimental.pallas import tpu as pltpu
from jax.experimental.pallas import tpu_sc as plsc

_NC, _NS = 2, 16
_NW = _NC * _NS
_CHUNK = 128


def _emb_body(idx_hbm, table_hbm, out_hbm, idx_v, rows0, rows1, sem0, sem1):
    wid = lax.axis_index("s") * _NC + lax.axis_index("c")
    n = idx_hbm.shape[1]
    pltpu.sync_copy(idx_hbm.at[wid], idx_v)

    rows = (rows0, rows1)
    sems = (sem0, sem1)

    pltpu.async_copy(table_hbm.at[idx_v.at[0]], rows0, sem0)

    def body(j, _):
        slot = lax.rem(j, 2)

        @pl.when(j + 1 < n)
        def _():
            nxt = lax.rem(j + 1, 2)
            for b in range(2):
                @pl.when(nxt == b)
                def _():
                    pltpu.async_copy(table_hbm.at[idx_v.at[j + 1]],
                                     rows[b], sems[b])

        for b in range(2):
            @pl.when(slot == b)
            def _():
                pltpu.make_async_copy(table_hbm.at[idx_v.at[j]],
                                      rows[b], sems[b]).wait()
                pltpu.sync_copy(
                    rows[b],
                    out_hbm.at[pl.ds((wid * n + j) * _CHUNK, _CHUNK)])
        return 0

    lax.fori_loop(0, n, body, 0)


def kernel(token_ids, lookup):
    bsz, seq = token_ids.shape
    num, dim = lookup.shape
    total = bsz * seq
    n = total // (_NW * _CHUNK)

    idx = token_ids.reshape(_NW, n, _CHUNK).astype(jnp.int32)

    call = functools.partial(
        pl.kernel,
        mesh=plsc.VectorSubcoreMesh(core_axis_name="c", subcore_axis_name="s"),
        out_type=jax.ShapeDtypeStruct((total, dim), jnp.float32),
        scratch_types=[
            pltpu.VMEM((n, _CHUNK), jnp.int32),
            pltpu.VMEM((_CHUNK, dim), jnp.float32),
            pltpu.VMEM((_CHUNK, dim), jnp.float32),
            pltpu.SemaphoreType.DMA,
            pltpu.SemaphoreType.DMA,
        ],
    )(_emb_body)

    out = call(idx, lookup)
    return out.reshape(bsz, seq, dim)


# speedup vs baseline: 3.3320x; 1.0003x over previous
import functools

import jax
import jax.numpy as jnp
from jax import lax
from jax.experimental import pallas as pl
from jax.experimental.pallas import tpu as pltpu
from jax.experimental.pallas import tpu_sc as plsc

_NC, _NS = 2, 16
_NW = _NC * _NS
_CHUNK = 128
_NBUF = 5
_D = 3


def _emb_body(idx_hbm, table_hbm, out_hbm, idx_v, *bufs):
    rows = bufs[:_NBUF]
    gsem = bufs[_NBUF:2 * _NBUF]
    wsem = bufs[2 * _NBUF:3 * _NBUF]

    wid = lax.axis_index("s") * _NC + lax.axis_index("c")
    n = idx_hbm.shape[1]
    pltpu.sync_copy(idx_hbm.at[wid], idx_v)

    def body(j, _):
        @pl.when(j < n)
        def _():
            slot = lax.rem(j, _NBUF)
            for b in range(_NBUF):
                @pl.when(slot == b)
                def _():
                    @pl.when(j >= _NBUF)
                    def _():
                        pltpu.make_async_copy(
                            rows[b],
                            out_hbm.at[pl.ds((wid * n + j - _NBUF) * _CHUNK,
                                             _CHUNK)],
                            wsem[b]).wait()
                    pltpu.async_copy(table_hbm.at[idx_v.at[j]],
                                     rows[b], gsem[b])

        i = j - _D
        @pl.when(i >= 0)
        def _():
            slot = lax.rem(i, _NBUF)
            for b in range(_NBUF):
                @pl.when(slot == b)
                def _():
                    pltpu.make_async_copy(table_hbm.at[idx_v.at[i]],
                                          rows[b], gsem[b]).wait()
                    pltpu.async_copy(
                        rows[b],
                        out_hbm.at[pl.ds((wid * n + i) * _CHUNK, _CHUNK)],
                        wsem[b])
        return 0

    lax.fori_loop(0, n + _D, body, 0)

    for b in range(_NBUF):
        chunk = n - _NBUF + ((b - n) % _NBUF)
        pltpu.make_async_copy(
            rows[b],
            out_hbm.at[pl.ds((wid * n + chunk) * _CHUNK, _CHUNK)],
            wsem[b]).wait()


def kernel(token_ids, lookup):
    bsz, seq = token_ids.shape
    num, dim = lookup.shape
    total = bsz * seq
    n = total // (_NW * _CHUNK)

    idx = token_ids.reshape(_NW, n, _CHUNK).astype(jnp.int32)

    call = functools.partial(
        pl.kernel,
        mesh=plsc.VectorSubcoreMesh(core_axis_name="c", subcore_axis_name="s"),
        out_type=jax.ShapeDtypeStruct((total, dim), jnp.float32),
        scratch_types=(
            [pltpu.VMEM((n, _CHUNK), jnp.int32)]
            + [pltpu.VMEM((_CHUNK, dim), jnp.float32) for _ in range(_NBUF)]
            + [pltpu.SemaphoreType.DMA for _ in range(2 * _NBUF)]
        ),
    )(_emb_body)

    out = call(idx, lookup)
    return out.reshape(bsz, seq, dim)
